# baseline (speedup 1.0000x reference)
import functools

import jax
import jax.numpy as jnp
from jax import lax
from jax.experimental import pallas as pl
from jax.experimental.pallas import tpu as pltpu
from jax.experimental.pallas import tpu_sc as plsc

NC, NS = 2, 16
NW = NC * NS
LANES = 16

B, L = 4096, 200
LBLK, BBLK = L // 8, B // 128
PAIRS = LBLK * BBLK
PAIRS_PER_W = PAIRS // NW
WD = 32
FV, FD = 1000, 16
PV, PD = 200, 16
CD = FD + 2 * PD
FS, PS = 1024, 256

_MESH = dict(core_axis_name="c", subcore_axis_name="s",
             num_cores=NC, num_subcores=NS)
_PARAMS = pltpu.CompilerParams(
    use_tc_tiling_on_sc=False, needs_layout_passes=False)


@jax.jit
def _sc_fp(f4, p4, r4, Wf2, Wp2, Wr2):
    @functools.partial(
        pl.kernel,
        mesh=plsc.VectorSubcoreMesh(**_MESH),
        out_type=[jax.ShapeDtypeStruct((L, CD // 8, BBLK, 8, 128), jnp.float32)],
        scratch_types=[
            pltpu.VMEM((2, 8, 128), jnp.int32),
            pltpu.VMEM((2, 8, 128), jnp.int32),
            pltpu.VMEM((2, 8, 128), jnp.int32),
            pltpu.VMEM((FD * FS,), jnp.float32),
            pltpu.VMEM((PD * PS,), jnp.float32),
            pltpu.VMEM((PD * PS,), jnp.float32),
            pltpu.VMEM((CD // 8, 8, 8, 128), jnp.float32),
            pltpu.SemaphoreType.DMA,
            pltpu.SemaphoreType.DMA,
        ],
        compiler_params=_PARAMS,
    )
    def k_fp(f4_h, p4_h, r4_h, Wf_h, Wp_h, Wr_h,
             fp5, fidx, pidx, ridx, tabf, tabp, tabr, fpT, sem_i, sem_w):
        wid = lax.axis_index("s") * NC + lax.axis_index("c")

        t_cps = []
        for dl in range(FD):
            t_cps.append(pltpu.async_copy(
                Wf_h.at[dl], tabf.at[pl.ds(dl * FS, FV)], sem_i))
        for dl in range(PD):
            t_cps.append(pltpu.async_copy(
                Wp_h.at[dl], tabp.at[pl.ds(dl * PS, PV)], sem_i))
            t_cps.append(pltpu.async_copy(
                Wr_h.at[dl], tabr.at[pl.ds(dl * PS, PV)], sem_i))
        for cp in t_cps:
            cp.wait()

        def fire_idx(pid, sel):
            lblk = pid // BBLK
            bblk = pid % BBLK
            pltpu.async_copy(f4_h.at[lblk, bblk], fidx.at[sel], sem_i)
            pltpu.async_copy(p4_h.at[lblk, bblk], pidx.at[sel], sem_i)
            pltpu.async_copy(r4_h.at[lblk, bblk], ridx.at[sel], sem_i)

        def drain_idx():
            for r in (fidx, pidx, ridx):
                pltpu.make_async_copy(f4_h.at[0, 0], r.at[0], sem_i).wait()

        def drain_writes():
            for dblk in range(CD // 8):
                pltpu.make_async_copy(
                    fpT.at[dblk], fp5.at[pl.ds(0, 8), 0, 0], sem_w).wait()

        fire_idx(wid * PAIRS_PER_W, 0)

        def pair_body(pp, carry):
            sel = pp % 2
            pid = wid * PAIRS_PER_W + pp
            lblk = pid // BBLK
            bblk = pid % BBLK

            drain_idx()
            nxt = jnp.where(pp + 1 < PAIRS_PER_W, pid + 1, pid)
            fire_idx(nxt, 1 - sel)

            def fp_lin(lin):
                for j in range(8):
                    fv = fidx[sel, lin, pl.ds(16 * j, 16)]
                    pv = pidx[sel, lin, pl.ds(16 * j, 16)]
                    rv = ridx[sel, lin, pl.ds(16 * j, 16)]
                    for dl in range(FD):
                        x = plsc.load_gather(tabf.at[pl.ds(dl * FS, FS)], [fv])
                        fpT[dl // 8, lin, dl % 8, pl.ds(16 * j, 16)] = x
                    for dl in range(PD):
                        x = plsc.load_gather(tabp.at[pl.ds(dl * PS, PS)], [pv])
                        d = FD + dl
                        fpT[d // 8, lin, d % 8, pl.ds(16 * j, 16)] = x
                    for dl in range(PD):
                        x = plsc.load_gather(tabr.at[pl.ds(dl * PS, PS)], [rv])
                        d = FD + PD + dl
                        fpT[d // 8, lin, d % 8, pl.ds(16 * j, 16)] = x

            @pl.when(pp > 0)
            def _():
                drain_writes()
            plsc.parallel_loop(0, 8, unroll=2)(fp_lin)
            for dblk in range(CD // 8):
                pltpu.async_copy(
                    fpT.at[dblk],
                    fp5.at[pl.ds(8 * lblk, 8), dblk, bblk], sem_w)
            return carry

        lax.fori_loop(0, PAIRS_PER_W, pair_body, 0)
        drain_idx()
        drain_writes()

    return k_fp(f4, p4, r4, Wf2, Wp2, Wr2)


@jax.jit
def _sc_word(w4, Wwd):
    @functools.partial(
        pl.kernel,
        mesh=plsc.VectorSubcoreMesh(**_MESH),
        out_type=[jax.ShapeDtypeStruct((L * WD * B,), jnp.float32)],
        scratch_types=[
            pltpu.VMEM((2, 8, 128), jnp.int32),
            pltpu.VMEM((1024, WD), jnp.float32),
            pltpu.VMEM((8 * WD * 128,), jnp.float32),
            pltpu.SemaphoreType.DMA,
            pltpu.SemaphoreType.DMA,
            pltpu.SemaphoreType.DMA,
        ],
        compiler_params=_PARAMS,
    )
    def k_word(w4_h, Ww_h, word5, widx, wrows, wT, sem_i, sem_g, sem_w):
        wid = lax.axis_index("s") * NC + lax.axis_index("c")
        iota = lax.iota(jnp.int32, LANES)
        clo = (iota // 8) * 1024 + (iota % 8) * 128
        chi = clo + 2 * 1024

        def fire_idx(pid, sel):
            lblk = pid // BBLK
            bblk = pid % BBLK
            pltpu.async_copy(w4_h.at[lblk, bblk], widx.at[sel], sem_i)

        def drain_idx():
            pltpu.make_async_copy(w4_h.at[0, 0], widx.at[0], sem_i).wait()

        def drain_writes():
            pltpu.make_async_copy(
                wT, word5.at[pl.ds(0, 8 * WD * 128)], sem_w).wait()

        fire_idx(wid * PAIRS_PER_W, 0)

        def pair_body(pp, carry):
            sel = pp % 2
            pid = wid * PAIRS_PER_W + pp
            lblk = pid // BBLK
            bblk = pid % BBLK

            drain_idx()
            g_cps = [
                pltpu.async_copy(
                    Ww_h.at[widx.at[sel, j]],
                    wrows.at[pl.ds(128 * j, 128)], sem_g)
                for j in range(8)
            ]
            nxt = jnp.where(pp + 1 < PAIRS_PER_W, pid + 1, pid)
            fire_idx(nxt, 1 - sel)

            for cp in g_cps:
                cp.wait()

            @pl.when(pp > 0)
            def _():
                drain_writes()

            def w_lin(lin):
                for bin_ in range(128):
                    tok = 128 * lin + bin_
                    xlo = wrows[tok, pl.ds(0, 16)]
                    xhi = wrows[tok, pl.ds(16, 16)]
                    base = lin * (WD * 128) + bin_
                    plsc.store_scatter(wT, [clo + base], xlo)
                    plsc.store_scatter(wT, [chi + base], xhi)

            plsc.parallel_loop(0, 8, unroll=2)(w_lin)

            wbase = (8 * lblk * (WD // 8) * BBLK + bblk) * 1024
            for lin in range(8):
                for dblk in range(WD // 8):
                    src = wT.at[pl.ds(lin * (WD * 128) + dblk * 1024, 1024)]
                    off = wbase + (lin * (WD // 8) * BBLK + dblk * BBLK) * 1024
                    pltpu.async_copy(src, word5.at[pl.ds(off, 1024)], sem_w)
            return carry

        lax.fori_loop(0, PAIRS_PER_W, pair_body, 0)
        drain_idx()
        drain_writes()

    return k_word(w4, Wwd)


def kernel(words, fields, pos, rpos, W_word, W_field, W_pos, W_rpos):
    def view4(ix):
        return ix.T.reshape(LBLK, 8, BBLK, 128).transpose(0, 2, 1, 3).astype(jnp.int32)

    (fp5,) = _sc_fp(view4(fields), view4(pos), view4(rpos),
                    W_field.T, W_pos.T, W_rpos.T)
    (word5,) = _sc_word(view4(words), W_word)
    word = (word5.reshape(L, WD // 8, BBLK, 8, 128)
            .transpose(2, 4, 0, 1, 3).reshape(B, L, WD))
    fp = fp5.transpose(2, 4, 0, 1, 3).reshape(B, L, CD)
    return word, fp


# speedup vs baseline: 1.0895x; 1.0895x over previous
import functools

import jax
import jax.numpy as jnp
from jax import lax
from jax.experimental import pallas as pl
from jax.experimental.pallas import tpu as pltpu
from jax.experimental.pallas import tpu_sc as plsc

NC, NS = 2, 16
NW = NC * NS
LANES = 16

B, L = 4096, 200
LBLK, BBLK = L // 8, B // 128
PAIRS = LBLK * BBLK
PAIRS_PER_W = PAIRS // NW
WD = 32
FV, FD = 1000, 16
PV, PD = 200, 16
CD = FD + 2 * PD
FS, PS = 1024, 256

_MESH = dict(core_axis_name="c", subcore_axis_name="s",
             num_cores=NC, num_subcores=NS)
_PARAMS = pltpu.CompilerParams(
    use_tc_tiling_on_sc=False, needs_layout_passes=False)


@jax.jit
def _sc_fp(f4, p4, r4, Wf2, Wp2, Wr2):
    @functools.partial(
        pl.kernel,
        mesh=plsc.VectorSubcoreMesh(**_MESH),
        out_type=[jax.ShapeDtypeStruct((L, CD // 8, BBLK, 8, 128), jnp.float32)],
        scratch_types=[
            pltpu.VMEM((2, 8, 128), jnp.int32),
            pltpu.VMEM((2, 8, 128), jnp.int32),
            pltpu.VMEM((2, 8, 128), jnp.int32),
            pltpu.VMEM((FD * FS,), jnp.float32),
            pltpu.VMEM((PD * PS,), jnp.float32),
            pltpu.VMEM((PD * PS,), jnp.float32),
            pltpu.VMEM((CD // 8, 8, 8, 128), jnp.float32),
            pltpu.SemaphoreType.DMA,
            pltpu.SemaphoreType.DMA,
        ],
        compiler_params=_PARAMS,
    )
    def k_fp(f4_h, p4_h, r4_h, Wf_h, Wp_h, Wr_h,
             fp5, fidx, pidx, ridx, tabf, tabp, tabr, fpT, sem_i, sem_w):
        wid = lax.axis_index("s") * NC + lax.axis_index("c")

        t_cps = []
        for dl in range(FD):
            t_cps.append(pltpu.async_copy(
                Wf_h.at[dl], tabf.at[pl.ds(dl * FS, FV)], sem_i))
        for dl in range(PD):
            t_cps.append(pltpu.async_copy(
                Wp_h.at[dl], tabp.at[pl.ds(dl * PS, PV)], sem_i))
            t_cps.append(pltpu.async_copy(
                Wr_h.at[dl], tabr.at[pl.ds(dl * PS, PV)], sem_i))
        for cp in t_cps:
            cp.wait()

        def fire_idx(pid, sel):
            lblk = pid // BBLK
            bblk = pid % BBLK
            pltpu.async_copy(f4_h.at[lblk, bblk], fidx.at[sel], sem_i)
            pltpu.async_copy(p4_h.at[lblk, bblk], pidx.at[sel], sem_i)
            pltpu.async_copy(r4_h.at[lblk, bblk], ridx.at[sel], sem_i)

        def drain_idx():
            for r in (fidx, pidx, ridx):
                pltpu.make_async_copy(f4_h.at[0, 0], r.at[0], sem_i).wait()

        def drain_writes():
            for dblk in range(CD // 8):
                pltpu.make_async_copy(
                    fpT.at[dblk], fp5.at[pl.ds(0, 8), 0, 0], sem_w).wait()

        fire_idx(wid * PAIRS_PER_W, 0)

        def pair_body(pp, carry):
            sel = pp % 2
            pid = wid * PAIRS_PER_W + pp
            lblk = pid // BBLK
            bblk = pid % BBLK

            drain_idx()
            nxt = jnp.where(pp + 1 < PAIRS_PER_W, pid + 1, pid)
            fire_idx(nxt, 1 - sel)

            def fp_lin(lin):
                for j in range(8):
                    fv = fidx[sel, lin, pl.ds(16 * j, 16)]
                    pv = pidx[sel, lin, pl.ds(16 * j, 16)]
                    rv = ridx[sel, lin, pl.ds(16 * j, 16)]
                    for dl in range(FD):
                        x = plsc.load_gather(tabf.at[pl.ds(dl * FS, FS)], [fv])
                        fpT[dl // 8, lin, dl % 8, pl.ds(16 * j, 16)] = x
                    for dl in range(PD):
                        x = plsc.load_gather(tabp.at[pl.ds(dl * PS, PS)], [pv])
                        d = FD + dl
                        fpT[d // 8, lin, d % 8, pl.ds(16 * j, 16)] = x
                    for dl in range(PD):
                        x = plsc.load_gather(tabr.at[pl.ds(dl * PS, PS)], [rv])
                        d = FD + PD + dl
                        fpT[d // 8, lin, d % 8, pl.ds(16 * j, 16)] = x

            @pl.when(pp > 0)
            def _():
                drain_writes()
            plsc.parallel_loop(0, 8)(fp_lin)
            for dblk in range(CD // 8):
                pltpu.async_copy(
                    fpT.at[dblk],
                    fp5.at[pl.ds(8 * lblk, 8), dblk, bblk], sem_w)
            return carry

        lax.fori_loop(0, PAIRS_PER_W, pair_body, 0)
        drain_idx()
        drain_writes()

    return k_fp(f4, p4, r4, Wf2, Wp2, Wr2)


@jax.jit
def _sc_word(w4, Wwd):
    @functools.partial(
        pl.kernel,
        mesh=plsc.VectorSubcoreMesh(**_MESH),
        out_type=[jax.ShapeDtypeStruct((L * WD * B,), jnp.float32)],
        scratch_types=[
            pltpu.VMEM((2, 8, 128), jnp.int32),
            pltpu.VMEM((1024, WD), jnp.float32),
            pltpu.VMEM((8 * WD * 128,), jnp.float32),
            pltpu.SemaphoreType.DMA,
            pltpu.SemaphoreType.DMA,
            pltpu.SemaphoreType.DMA,
        ],
        compiler_params=_PARAMS,
    )
    def k_word(w4_h, Ww_h, word5, widx, wrows, wT, sem_i, sem_g, sem_w):
        wid = lax.axis_index("s") * NC + lax.axis_index("c")
        iota = lax.iota(jnp.int32, LANES)
        clo = (iota // 8) * 1024 + (iota % 8) * 128
        chi = clo + 2 * 1024

        def fire_idx(pid, sel):
            lblk = pid // BBLK
            bblk = pid % BBLK
            pltpu.async_copy(w4_h.at[lblk, bblk], widx.at[sel], sem_i)

        def drain_idx():
            pltpu.make_async_copy(w4_h.at[0, 0], widx.at[0], sem_i).wait()

        def drain_writes():
            pltpu.make_async_copy(
                wT, word5.at[pl.ds(0, 8 * WD * 128)], sem_w).wait()

        fire_idx(wid * PAIRS_PER_W, 0)

        def pair_body(pp, carry):
            sel = pp % 2
            pid = wid * PAIRS_PER_W + pp
            lblk = pid // BBLK
            bblk = pid % BBLK

            drain_idx()
            g_cps = [
                pltpu.async_copy(
                    Ww_h.at[widx.at[sel, j]],
                    wrows.at[pl.ds(128 * j, 128)], sem_g)
                for j in range(8)
            ]
            nxt = jnp.where(pp + 1 < PAIRS_PER_W, pid + 1, pid)
            fire_idx(nxt, 1 - sel)

            for cp in g_cps:
                cp.wait()

            @pl.when(pp > 0)
            def _():
                drain_writes()

            def w_lin(lin):
                for bin_ in range(128):
                    tok = 128 * lin + bin_
                    xlo = wrows[tok, pl.ds(0, 16)]
                    xhi = wrows[tok, pl.ds(16, 16)]
                    base = lin * (WD * 128) + bin_
                    plsc.store_scatter(wT, [clo + base], xlo)
                    plsc.store_scatter(wT, [chi + base], xhi)

            plsc.parallel_loop(0, 8)(w_lin)

            wbase = (8 * lblk * (WD // 8) * BBLK + bblk) * 1024
            for lin in range(8):
                for dblk in range(WD // 8):
                    src = wT.at[pl.ds(lin * (WD * 128) + dblk * 1024, 1024)]
                    off = wbase + (lin * (WD // 8) * BBLK + dblk * BBLK) * 1024
                    pltpu.async_copy(src, word5.at[pl.ds(off, 1024)], sem_w)
            return carry

        lax.fori_loop(0, PAIRS_PER_W, pair_body, 0)
        drain_idx()
        drain_writes()

    return k_word(w4, Wwd)


def kernel(words, fields, pos, rpos, W_word, W_field, W_pos, W_rpos):
    def view4(ix):
        return ix.T.reshape(LBLK, 8, BBLK, 128).transpose(0, 2, 1, 3).astype(jnp.int32)

    (fp5,) = _sc_fp(view4(fields), view4(pos), view4(rpos),
                    W_field.T, W_pos.T, W_rpos.T)
    (word5,) = _sc_word(view4(words), W_word)
    word = (word5.reshape(L, WD // 8, BBLK, 8, 128)
            .transpose(2, 4, 0, 1, 3).reshape(B, L, WD))
    fp = fp5.transpose(2, 4, 0, 1, 3).reshape(B, L, CD)
    return word, fp
